# baseline (speedup 1.0000x reference)
import functools

import jax
import jax.numpy as jnp
from jax import lax
from jax.experimental import pallas as pl
from jax.experimental.pallas import tpu as pltpu
from jax.experimental.pallas import tpu_sc as plsc

NC = 2
NS = 16
NW = NC * NS
B = 128


def _make_sc_deg(N, C, n_pad):
    stripe = n_pad // NS

    mesh = plsc.VectorSubcoreMesh(core_axis_name="c", subcore_axis_name="s")

    @functools.partial(
        pl.kernel,
        mesh=mesh,
        out_type=jax.ShapeDtypeStruct((NC, n_pad), jnp.float32),
        scratch_types=[
            pltpu.VMEM((C, B), jnp.int32),
            pltpu.VMEM((B,), jnp.float32),
            pltpu.VMEM((stripe,), jnp.float32),
            pltpu.VMEM_SHARED((n_pad,), jnp.float32),
        ],
    )
    def deg_kernel(row_hbm, degp_hbm, idx_v, ones_v, zero_v, acc_sh):
        c = lax.axis_index("c")
        s = lax.axis_index("s")
        w = c * NS + s

        @pl.loop(0, stripe // 16)
        def _(i):
            zero_v[pl.ds(i * 16, 16)] = jnp.zeros((16,), jnp.float32)

        @pl.loop(0, B // 16)
        def _(j):
            ones_v[pl.ds(j * 16, 16)] = jnp.ones((16,), jnp.float32)

        pltpu.sync_copy(zero_v, acc_sh.at[pl.ds(s * stripe, stripe)])
        plsc.subcore_barrier()

        pltpu.sync_copy(row_hbm.at[w], idx_v)

        @pl.loop(0, C)
        def _(j):
            pltpu.sync_copy(ones_v, acc_sh.at[idx_v.at[j]], add=True)

        plsc.subcore_barrier()
        pltpu.sync_copy(
            acc_sh.at[pl.ds(s * stripe, stripe)],
            degp_hbm.at[c, pl.ds(s * stripe, stripe)],
        )

    return deg_kernel


def _make_sc_agg(N, D, C, half, rows_pad):
    stripe = rows_pad // NS
    zr = 80
    NBUF = 2

    mesh = plsc.VectorSubcoreMesh(core_axis_name="c", subcore_axis_name="s")

    @functools.partial(
        pl.kernel,
        mesh=mesh,
        out_type=jax.ShapeDtypeStruct((NC, rows_pad, D), jnp.float32),
        scratch_types=[
            pltpu.VMEM((C, B), jnp.int32),
            pltpu.VMEM((C, B), jnp.int32),
            pltpu.VMEM((NBUF, B, D), jnp.float32),
            pltpu.VMEM((zr, D), jnp.float32),
            pltpu.VMEM_SHARED((rows_pad, D), jnp.float32),
        ] + [pltpu.SemaphoreType.DMA] * (2 * NBUF),
    )
    def agg_kernel(g_hbm, rowg_hbm, cols_hbm, accp_hbm,
                   rowi_v, coli_v, rows_v, zero_v, acc_sh, *sems):
        gsems = sems[:NBUF]
        ssems = sems[NBUF:]
        c = lax.axis_index("c")
        s = lax.axis_index("s")

        @pl.loop(0, zr)
        def _(i):
            for k in range(D // 16):
                zero_v[i, pl.ds(k * 16, 16)] = jnp.zeros((16,), jnp.float32)

        base = s * stripe
        off = 0
        while off < stripe:
            n = min(zr, stripe - off)
            pltpu.sync_copy(zero_v.at[pl.ds(0, n)],
                            acc_sh.at[pl.ds(base + off, n)])
            off += n

        pltpu.sync_copy(rowg_hbm.at[s], rowi_v)
        pltpu.sync_copy(cols_hbm.at[s], coli_v)

        node0 = c * half

        @pl.loop(0, C)
        def _(j):
            for k in range(B // 16):
                v = coli_v[j, pl.ds(k * 16, 16)]
                lv = v - node0
                ok = (lv >= 0) & (lv < half)
                coli_v[j, pl.ds(k * 16, 16)] = jnp.where(ok, lv, half)

        plsc.subcore_barrier()

        @pl.loop(0, C)
        def _(j):
            pltpu.sync_copy(g_hbm.at[rowi_v.at[j]], rows_v.at[0])
            pltpu.sync_copy(rows_v.at[0], acc_sh.at[coli_v.at[j]], add=True)

        plsc.subcore_barrier()
        pltpu.sync_copy(
            acc_sh.at[pl.ds(s * stripe, stripe)],
            accp_hbm.at[c, pl.ds(s * stripe, stripe)],
        )

    return agg_kernel


def _tc_pre_body(x_ref, w_ref, b_ref, dp_ref, g_ref, dis_ref):
    h = lax.dot_general(
        x_ref[...], w_ref[...],
        (((1,), (1,)), ((), ())),
        preferred_element_type=jnp.float32,
    ) + b_ref[...]
    deg = dp_ref[0, :] + dp_ref[1, :]
    dis = lax.rsqrt(deg)
    g_ref[...] = dis[:, None] * h
    dis_ref[...] = dis[None, :]


def _tc_post_body(acc_ref, dis_ref, out_ref):
    out_ref[...] = dis_ref[0, :][:, None] * acc_ref[...]


def kernel(x, edge_index, W, b):
    N, D = x.shape
    E = edge_index.shape[1]
    row = edge_index[0]
    col = edge_index[1]

    Cd = -(-E // (NW * B))
    pad_d = NW * Cd * B - E
    C = -(-(-(-E // (NS * B))) // 4) * 4
    pad_a = NS * C * B - E
    n_pad = -(-N // (16 * NS)) * (16 * NS)
    half = -(-N // 2)
    rows_pad = -(-(half + 1) // (8 * NS)) * (8 * NS)

    rowd = jnp.concatenate(
        [row, jnp.full((pad_d,), N, jnp.int32)]).reshape(NW, Cd, B)
    rowg = jnp.concatenate(
        [row, jnp.zeros((pad_a,), jnp.int32)]).reshape(NS, C, B)
    cols = jnp.concatenate(
        [col, jnp.full((pad_a,), N, jnp.int32)]).reshape(NS, C, B)

    degp = _make_sc_deg(N, Cd, n_pad)(rowd)
    dp = degp[:, :N]

    BN = 512
    grid = (-(-N // BN),)
    g, dis = pl.pallas_call(
        _tc_pre_body,
        grid=grid,
        in_specs=[
            pl.BlockSpec((BN, D), lambda i: (i, 0)),
            pl.BlockSpec((D, D), lambda i: (0, 0)),
            pl.BlockSpec((1, D), lambda i: (0, 0)),
            pl.BlockSpec((2, BN), lambda i: (0, i)),
        ],
        out_specs=[
            pl.BlockSpec((BN, D), lambda i: (i, 0)),
            pl.BlockSpec((1, BN), lambda i: (0, i)),
        ],
        out_shape=[
            jax.ShapeDtypeStruct((N, D), jnp.float32),
            jax.ShapeDtypeStruct((1, N), jnp.float32),
        ],
    )(x, W, b[None, :], dp)

    accp = _make_sc_agg(N, D, C, half, rows_pad)(g, rowg, cols)
    acc = jnp.concatenate([accp[0, :half], accp[1, :N - half]], axis=0)

    out = pl.pallas_call(
        _tc_post_body,
        grid=grid,
        in_specs=[
            pl.BlockSpec((BN, D), lambda i: (i, 0)),
            pl.BlockSpec((1, BN), lambda i: (0, i)),
        ],
        out_specs=pl.BlockSpec((BN, D), lambda i: (i, 0)),
        out_shape=jax.ShapeDtypeStruct((N, D), jnp.float32),
    )(acc, dis)
    return out


# speedup vs baseline: 1.0002x; 1.0002x over previous
import functools

import jax
import jax.numpy as jnp
from jax import lax
from jax.experimental import pallas as pl
from jax.experimental.pallas import tpu as pltpu
from jax.experimental.pallas import tpu_sc as plsc

NC = 2
NS = 16
NW = NC * NS
B = 128


def _make_sc_deg(N, C, n_pad):
    stripe = n_pad // NS

    mesh = plsc.VectorSubcoreMesh(core_axis_name="c", subcore_axis_name="s")

    @functools.partial(
        pl.kernel,
        mesh=mesh,
        out_type=jax.ShapeDtypeStruct((NC, n_pad), jnp.float32),
        scratch_types=[
            pltpu.VMEM((C, B), jnp.int32),
            pltpu.VMEM((B,), jnp.float32),
            pltpu.VMEM((stripe,), jnp.float32),
            pltpu.VMEM_SHARED((n_pad,), jnp.float32),
        ],
    )
    def deg_kernel(row_hbm, degp_hbm, idx_v, ones_v, zero_v, acc_sh):
        c = lax.axis_index("c")
        s = lax.axis_index("s")
        w = c * NS + s

        @pl.loop(0, stripe // 16)
        def _(i):
            zero_v[pl.ds(i * 16, 16)] = jnp.zeros((16,), jnp.float32)

        @pl.loop(0, B // 16)
        def _(j):
            ones_v[pl.ds(j * 16, 16)] = jnp.ones((16,), jnp.float32)

        pltpu.sync_copy(zero_v, acc_sh.at[pl.ds(s * stripe, stripe)])
        plsc.subcore_barrier()

        pltpu.sync_copy(row_hbm.at[w], idx_v)

        @pl.loop(0, C)
        def _(j):
            pltpu.sync_copy(ones_v, acc_sh.at[idx_v.at[j]], add=True)

        plsc.subcore_barrier()
        pltpu.sync_copy(
            acc_sh.at[pl.ds(s * stripe, stripe)],
            degp_hbm.at[c, pl.ds(s * stripe, stripe)],
        )

    return deg_kernel


def _make_sc_agg(N, D, C, half, rows_pad):
    stripe = rows_pad // NS
    zr = 80
    NBUF = 2

    mesh = plsc.VectorSubcoreMesh(core_axis_name="c", subcore_axis_name="s")

    @functools.partial(
        pl.kernel,
        mesh=mesh,
        out_type=jax.ShapeDtypeStruct((NC, rows_pad, D), jnp.float32),
        scratch_types=[
            pltpu.VMEM((C, B), jnp.int32),
            pltpu.VMEM((C, B), jnp.int32),
            pltpu.VMEM((B, D), jnp.float32),
            pltpu.VMEM((zr, D), jnp.float32),
            pltpu.VMEM_SHARED((rows_pad, D), jnp.float32),
            pltpu.SemaphoreType.DMA,
        ],
    )
    def agg_kernel(g_hbm, rowg_hbm, cols_hbm, accp_hbm,
                   rowi_v, coli_v, rows_v, zero_v, acc_sh, sem):
        c = lax.axis_index("c")
        s = lax.axis_index("s")

        @pl.loop(0, zr)
        def _(i):
            for k in range(D // 16):
                zero_v[i, pl.ds(k * 16, 16)] = jnp.zeros((16,), jnp.float32)

        base = s * stripe
        off = 0
        while off < stripe:
            n = min(zr, stripe - off)
            pltpu.sync_copy(zero_v.at[pl.ds(0, n)],
                            acc_sh.at[pl.ds(base + off, n)])
            off += n

        pltpu.sync_copy(rowg_hbm.at[s], rowi_v)
        pltpu.sync_copy(cols_hbm.at[s], coli_v)

        node0 = c * half

        @pl.loop(0, C)
        def _(j):
            for k in range(B // 16):
                v = coli_v[j, pl.ds(k * 16, 16)]
                lv = v - node0
                ok = (lv >= 0) & (lv < half)
                coli_v[j, pl.ds(k * 16, 16)] = jnp.where(ok, lv, half)

        plsc.subcore_barrier()

        @pl.loop(0, C)
        def _(j):
            pltpu.async_copy(g_hbm.at[rowi_v.at[j]], rows_v, sem).wait()
            pltpu.sync_copy(rows_v, acc_sh.at[coli_v.at[j]], add=True)

        plsc.subcore_barrier()
        pltpu.sync_copy(
            acc_sh.at[pl.ds(s * stripe, stripe)],
            accp_hbm.at[c, pl.ds(s * stripe, stripe)],
        )

    return agg_kernel


def _tc_pre_body(x_ref, w_ref, b_ref, dp_ref, g_ref, dis_ref):
    h = lax.dot_general(
        x_ref[...], w_ref[...],
        (((1,), (1,)), ((), ())),
        preferred_element_type=jnp.float32,
    ) + b_ref[...]
    deg = dp_ref[0, :] + dp_ref[1, :]
    dis = lax.rsqrt(deg)
    g_ref[...] = dis[:, None] * h
    dis_ref[...] = dis[None, :]


def _tc_post_body(acc_ref, dis_ref, out_ref):
    out_ref[...] = dis_ref[0, :][:, None] * acc_ref[...]


def kernel(x, edge_index, W, b):
    N, D = x.shape
    E = edge_index.shape[1]
    row = edge_index[0]
    col = edge_index[1]

    Cd = -(-E // (NW * B))
    pad_d = NW * Cd * B - E
    C = -(-(-(-E // (NS * B))) // 4) * 4
    pad_a = NS * C * B - E
    n_pad = -(-N // (16 * NS)) * (16 * NS)
    half = -(-N // 2)
    rows_pad = -(-(half + 1) // (8 * NS)) * (8 * NS)

    rowd = jnp.concatenate(
        [row, jnp.full((pad_d,), N, jnp.int32)]).reshape(NW, Cd, B)
    rowg = jnp.concatenate(
        [row, jnp.zeros((pad_a,), jnp.int32)]).reshape(NS, C, B)
    cols = jnp.concatenate(
        [col, jnp.full((pad_a,), N, jnp.int32)]).reshape(NS, C, B)

    degp = _make_sc_deg(N, Cd, n_pad)(rowd)
    dp = degp[:, :N]

    BN = 512
    grid = (-(-N // BN),)
    g, dis = pl.pallas_call(
        _tc_pre_body,
        grid=grid,
        in_specs=[
            pl.BlockSpec((BN, D), lambda i: (i, 0)),
            pl.BlockSpec((D, D), lambda i: (0, 0)),
            pl.BlockSpec((1, D), lambda i: (0, 0)),
            pl.BlockSpec((2, BN), lambda i: (0, i)),
        ],
        out_specs=[
            pl.BlockSpec((BN, D), lambda i: (i, 0)),
            pl.BlockSpec((1, BN), lambda i: (0, i)),
        ],
        out_shape=[
            jax.ShapeDtypeStruct((N, D), jnp.float32),
            jax.ShapeDtypeStruct((1, N), jnp.float32),
        ],
    )(x, W, b[None, :], dp)

    accp = _make_sc_agg(N, D, C, half, rows_pad)(g, rowg, cols)
    acc = jnp.concatenate([accp[0, :half], accp[1, :N - half]], axis=0)

    out = pl.pallas_call(
        _tc_post_body,
        grid=grid,
        in_specs=[
            pl.BlockSpec((BN, D), lambda i: (i, 0)),
            pl.BlockSpec((1, BN), lambda i: (0, i)),
        ],
        out_specs=pl.BlockSpec((BN, D), lambda i: (i, 0)),
        out_shape=jax.ShapeDtypeStruct((N, D), jnp.float32),
    )(acc, dis)
    return out


# speedup vs baseline: 1.8204x; 1.8200x over previous
import functools

import jax
import jax.numpy as jnp
from jax import lax
from jax.experimental import pallas as pl
from jax.experimental.pallas import tpu as pltpu
from jax.experimental.pallas import tpu_sc as plsc

NC = 2
NS = 16
NW = NC * NS
B = 128


def _make_sc_deg(N, C, n_pad):
    stripe = n_pad // NS

    mesh = plsc.VectorSubcoreMesh(core_axis_name="c", subcore_axis_name="s")

    @functools.partial(
        pl.kernel,
        mesh=mesh,
        out_type=jax.ShapeDtypeStruct((NC, n_pad), jnp.float32),
        scratch_types=[
            pltpu.VMEM((C, B), jnp.int32),
            pltpu.VMEM((B,), jnp.float32),
            pltpu.VMEM((stripe,), jnp.float32),
            pltpu.VMEM_SHARED((n_pad,), jnp.float32),
        ],
    )
    def deg_kernel(row_hbm, degp_hbm, idx_v, ones_v, zero_v, acc_sh):
        c = lax.axis_index("c")
        s = lax.axis_index("s")
        w = c * NS + s

        @pl.loop(0, stripe // 16)
        def _(i):
            zero_v[pl.ds(i * 16, 16)] = jnp.zeros((16,), jnp.float32)

        @pl.loop(0, B // 16)
        def _(j):
            ones_v[pl.ds(j * 16, 16)] = jnp.ones((16,), jnp.float32)

        pltpu.sync_copy(zero_v, acc_sh.at[pl.ds(s * stripe, stripe)])
        plsc.subcore_barrier()

        pltpu.sync_copy(row_hbm.at[w], idx_v)

        @pl.loop(0, C)
        def _(j):
            pltpu.sync_copy(ones_v, acc_sh.at[idx_v.at[j]], add=True)

        plsc.subcore_barrier()
        pltpu.sync_copy(
            acc_sh.at[pl.ds(s * stripe, stripe)],
            degp_hbm.at[c, pl.ds(s * stripe, stripe)],
        )

    return deg_kernel


def _make_sc_agg(N, D, C, half, rows_pad):
    stripe = rows_pad // NS
    zr = 160

    mesh = plsc.VectorSubcoreMesh(core_axis_name="c", subcore_axis_name="s")

    @functools.partial(
        pl.kernel,
        mesh=mesh,
        out_type=jax.ShapeDtypeStruct((NC, rows_pad, D), jnp.float32),
        scratch_types=[
            pltpu.VMEM((C, B), jnp.int32),
            pltpu.VMEM((C, B), jnp.int32),
            pltpu.VMEM((B, D), jnp.float32),
            pltpu.VMEM((zr, D), jnp.float32),
            pltpu.VMEM_SHARED((rows_pad, D), jnp.float32),
            pltpu.SemaphoreType.DMA,
        ],
    )
    def agg_kernel(g_hbm, rowg_hbm, cols_hbm, accp_hbm,
                   rowi_v, coli_v, rows_v, zero_v, acc_sh, sem):
        c = lax.axis_index("c")
        s = lax.axis_index("s")

        @pl.loop(0, zr)
        def _(i):
            for k in range(D // 16):
                zero_v[i, pl.ds(k * 16, 16)] = jnp.zeros((16,), jnp.float32)

        base = s * stripe
        off = 0
        while off < stripe:
            n = min(zr, stripe - off)
            pltpu.sync_copy(zero_v.at[pl.ds(0, n)],
                            acc_sh.at[pl.ds(base + off, n)])
            off += n

        pltpu.sync_copy(rowg_hbm.at[s], rowi_v)
        pltpu.sync_copy(cols_hbm.at[s], coli_v)

        node0 = c * half

        @pl.loop(0, C)
        def _(j):
            for k in range(B // 16):
                v = coli_v[j, pl.ds(k * 16, 16)]
                lv = v - node0
                ok = (lv >= 0) & (lv < half)
                coli_v[j, pl.ds(k * 16, 16)] = jnp.where(ok, lv, half)

        plsc.subcore_barrier()

        @pl.loop(0, C)
        def _(j):
            pltpu.async_copy(g_hbm.at[rowi_v.at[j]], rows_v, sem).wait()
            pltpu.sync_copy(rows_v, acc_sh.at[coli_v.at[j]], add=True)

        plsc.subcore_barrier()
        pltpu.sync_copy(
            acc_sh.at[pl.ds(s * stripe, stripe)],
            accp_hbm.at[c, pl.ds(s * stripe, stripe)],
        )

    return agg_kernel


def _tc_pre_body(x_ref, w_ref, b_ref, dp_ref, g_ref, dis_ref):
    h = lax.dot_general(
        x_ref[...], w_ref[...],
        (((1,), (1,)), ((), ())),
        preferred_element_type=jnp.float32,
    ) + b_ref[...]
    deg = dp_ref[0, :] + dp_ref[1, :]
    dis = lax.rsqrt(deg)
    g_ref[...] = dis[:, None] * h
    dis_ref[...] = dis[None, :]


def _tc_post_body(acc_ref, dis_ref, out_ref):
    out_ref[...] = dis_ref[0, :][:, None] * acc_ref[...]


def kernel(x, edge_index, W, b):
    N, D = x.shape
    E = edge_index.shape[1]
    row = edge_index[0]
    col = edge_index[1]

    Cd = -(-E // (NW * B))
    pad_d = NW * Cd * B - E
    C = -(-E // (NS * B))
    pad_a = NS * C * B - E
    n_pad = -(-N // (16 * NS)) * (16 * NS)
    half = -(-N // 2)
    rows_pad = -(-(half + 1) // (8 * NS)) * (8 * NS)

    rowd = jnp.concatenate(
        [row, jnp.full((pad_d,), N, jnp.int32)]).reshape(NW, Cd, B)
    rowg = jnp.concatenate(
        [row, jnp.zeros((pad_a,), jnp.int32)]).reshape(NS, C, B)
    cols = jnp.concatenate(
        [col, jnp.full((pad_a,), N, jnp.int32)]).reshape(NS, C, B)

    degp = _make_sc_deg(N, Cd, n_pad)(rowd)
    dp = degp[:, :N]

    BN = 512
    grid = (-(-N // BN),)
    g, dis = pl.pallas_call(
        _tc_pre_body,
        grid=grid,
        in_specs=[
            pl.BlockSpec((BN, D), lambda i: (i, 0)),
            pl.BlockSpec((D, D), lambda i: (0, 0)),
            pl.BlockSpec((1, D), lambda i: (0, 0)),
            pl.BlockSpec((2, BN), lambda i: (0, i)),
        ],
        out_specs=[
            pl.BlockSpec((BN, D), lambda i: (i, 0)),
            pl.BlockSpec((1, BN), lambda i: (0, i)),
        ],
        out_shape=[
            jax.ShapeDtypeStruct((N, D), jnp.float32),
            jax.ShapeDtypeStruct((1, N), jnp.float32),
        ],
    )(x, W, b[None, :], dp)

    accp = _make_sc_agg(N, D, C, half, rows_pad)(g, rowg, cols)
    acc = jnp.concatenate([accp[0, :half], accp[1, :N - half]], axis=0)

    out = pl.pallas_call(
        _tc_post_body,
        grid=grid,
        in_specs=[
            pl.BlockSpec((BN, D), lambda i: (i, 0)),
            pl.BlockSpec((1, BN), lambda i: (0, i)),
        ],
        out_specs=pl.BlockSpec((BN, D), lambda i: (i, 0)),
        out_shape=jax.ShapeDtypeStruct((N, D), jnp.float32),
    )(acc, dis)
    return out


# speedup vs baseline: 1.9002x; 1.0439x over previous
import functools

import jax
import jax.numpy as jnp
from jax import lax
from jax.experimental import pallas as pl
from jax.experimental.pallas import tpu as pltpu
from jax.experimental.pallas import tpu_sc as plsc

NC = 2
NS = 16
NW = NC * NS
B = 128


def _make_sc_deg(N, C, n_pad):
    stripe = n_pad // NS

    mesh = plsc.VectorSubcoreMesh(core_axis_name="c", subcore_axis_name="s")

    @functools.partial(
        pl.kernel,
        mesh=mesh,
        out_type=jax.ShapeDtypeStruct((NC, n_pad), jnp.float32),
        scratch_types=[
            pltpu.VMEM((C, B), jnp.int32),
            pltpu.VMEM((B,), jnp.float32),
            pltpu.VMEM((stripe,), jnp.float32),
            pltpu.VMEM_SHARED((n_pad,), jnp.float32),
        ],
    )
    def deg_kernel(row_hbm, degp_hbm, idx_v, ones_v, zero_v, acc_sh):
        c = lax.axis_index("c")
        s = lax.axis_index("s")
        w = c * NS + s

        @pl.loop(0, stripe // 16)
        def _(i):
            zero_v[pl.ds(i * 16, 16)] = jnp.zeros((16,), jnp.float32)

        @pl.loop(0, B // 16)
        def _(j):
            ones_v[pl.ds(j * 16, 16)] = jnp.ones((16,), jnp.float32)

        pltpu.sync_copy(zero_v, acc_sh.at[pl.ds(s * stripe, stripe)])
        plsc.subcore_barrier()

        pltpu.sync_copy(row_hbm.at[w], idx_v)

        @pl.loop(0, C)
        def _(j):
            pltpu.sync_copy(ones_v, acc_sh.at[idx_v.at[j]], add=True)

        plsc.subcore_barrier()
        pltpu.sync_copy(
            acc_sh.at[pl.ds(s * stripe, stripe)],
            degp_hbm.at[c, pl.ds(s * stripe, stripe)],
        )

    return deg_kernel


def _make_sc_agg(N, D, C, half, rows_pad):
    stripe = rows_pad // NS
    zr = 160

    mesh = plsc.VectorSubcoreMesh(core_axis_name="c", subcore_axis_name="s")

    @functools.partial(
        pl.kernel,
        mesh=mesh,
        out_type=jax.ShapeDtypeStruct((NC, rows_pad, D), jnp.float32),
        scratch_types=[
            pltpu.VMEM((C, B), jnp.int32),
            pltpu.VMEM((C, B), jnp.int32),
            pltpu.VMEM((B, D), jnp.float32),
            pltpu.VMEM_SHARED((rows_pad, D), jnp.float32),
            pltpu.SemaphoreType.DMA,
            pltpu.VMEM((B, D), jnp.float32),
        ],
    )
    def agg_kernel(g_hbm, rowg_hbm, cols_hbm, accp_hbm,
                   rowi_v, coli_v, rows_v, acc_sh, sem, rows_v2):
        c = lax.axis_index("c")
        s = lax.axis_index("s")

        @pl.loop(0, B)
        def _(i):
            for k in range(D // 16):
                rows_v[i, pl.ds(k * 16, 16)] = jnp.zeros((16,), jnp.float32)

        base = s * stripe
        off = 0
        while off < stripe:
            n = min(B, stripe - off)
            pltpu.sync_copy(rows_v.at[pl.ds(0, n)],
                            acc_sh.at[pl.ds(base + off, n)])
            off += n

        pltpu.sync_copy(rowg_hbm.at[s], rowi_v)
        pltpu.sync_copy(cols_hbm.at[s], coli_v)

        node0 = c * half

        @pl.loop(0, C)
        def _(j):
            for k in range(B // 16):
                v = coli_v[j, pl.ds(k * 16, 16)]
                lv = v - node0
                ok = (lv >= 0) & (lv < half)
                coli_v[j, pl.ds(k * 16, 16)] = jnp.where(ok, lv, half)

        plsc.subcore_barrier()

        @pl.loop(0, C // 2)
        def _(p):
            j = 2 * p
            d0 = pltpu.async_copy(g_hbm.at[rowi_v.at[j]], rows_v, sem)
            d1 = pltpu.async_copy(g_hbm.at[rowi_v.at[j + 1]], rows_v2, sem)
            d0.wait()
            d1.wait()
            s0 = pltpu.async_copy(rows_v, acc_sh.at[coli_v.at[j]], sem,
                                  add=True)
            s1 = pltpu.async_copy(rows_v2, acc_sh.at[coli_v.at[j + 1]],
                                  sem, add=True)
            s0.wait()
            s1.wait()

        if C % 2:
            jt = C - 1
            pltpu.async_copy(g_hbm.at[rowi_v.at[jt]], rows_v, sem).wait()
            pltpu.sync_copy(rows_v, acc_sh.at[coli_v.at[jt]], add=True)

        plsc.subcore_barrier()
        pltpu.sync_copy(
            acc_sh.at[pl.ds(s * stripe, stripe)],
            accp_hbm.at[c, pl.ds(s * stripe, stripe)],
        )

    return agg_kernel


def _tc_pre_body(x_ref, w_ref, b_ref, dp_ref, g_ref, dis_ref):
    h = lax.dot_general(
        x_ref[...], w_ref[...],
        (((1,), (1,)), ((), ())),
        preferred_element_type=jnp.float32,
    ) + b_ref[...]
    deg = dp_ref[0, :] + dp_ref[1, :]
    dis = lax.rsqrt(deg)
    g_ref[...] = dis[:, None] * h
    dis_ref[...] = dis[None, :]


def _tc_post_body(acc_ref, dis_ref, out_ref):
    out_ref[...] = dis_ref[0, :][:, None] * acc_ref[...]


def kernel(x, edge_index, W, b):
    N, D = x.shape
    E = edge_index.shape[1]
    row = edge_index[0]
    col = edge_index[1]

    Cd = -(-E // (NW * B))
    pad_d = NW * Cd * B - E
    C = -(-E // (NS * B))
    pad_a = NS * C * B - E
    n_pad = -(-N // (16 * NS)) * (16 * NS)
    half = -(-N // 2)
    rows_pad = -(-(half + 1) // (8 * NS)) * (8 * NS)

    rowd = jnp.concatenate(
        [row, jnp.full((pad_d,), N, jnp.int32)]).reshape(NW, Cd, B)
    rowg = jnp.concatenate(
        [row, jnp.zeros((pad_a,), jnp.int32)]).reshape(NS, C, B)
    cols = jnp.concatenate(
        [col, jnp.full((pad_a,), N, jnp.int32)]).reshape(NS, C, B)

    degp = _make_sc_deg(N, Cd, n_pad)(rowd)
    dp = degp[:, :N]

    BN = 512
    grid = (-(-N // BN),)
    g, dis = pl.pallas_call(
        _tc_pre_body,
        grid=grid,
        in_specs=[
            pl.BlockSpec((BN, D), lambda i: (i, 0)),
            pl.BlockSpec((D, D), lambda i: (0, 0)),
            pl.BlockSpec((1, D), lambda i: (0, 0)),
            pl.BlockSpec((2, BN), lambda i: (0, i)),
        ],
        out_specs=[
            pl.BlockSpec((BN, D), lambda i: (i, 0)),
            pl.BlockSpec((1, BN), lambda i: (0, i)),
        ],
        out_shape=[
            jax.ShapeDtypeStruct((N, D), jnp.float32),
            jax.ShapeDtypeStruct((1, N), jnp.float32),
        ],
    )(x, W, b[None, :], dp)

    accp = _make_sc_agg(N, D, C, half, rows_pad)(g, rowg, cols)
    acc = jnp.concatenate([accp[0, :half], accp[1, :N - half]], axis=0)

    out = pl.pallas_call(
        _tc_post_body,
        grid=grid,
        in_specs=[
            pl.BlockSpec((BN, D), lambda i: (i, 0)),
            pl.BlockSpec((1, BN), lambda i: (0, i)),
        ],
        out_specs=pl.BlockSpec((BN, D), lambda i: (i, 0)),
        out_shape=jax.ShapeDtypeStruct((N, D), jnp.float32),
    )(acc, dis)
    return out
